# Initial kernel scaffold; baseline (speedup 1.0000x reference)
import jax
import jax.numpy as jnp
from jax.experimental import pallas as pl


def kernel(x, pos_table):
    raise NotImplementedError("write your pallas kernel here")


# speedup vs baseline: 1.4955x; 1.4955x over previous
import jax
import jax.numpy as jnp
from jax.experimental import pallas as pl


_SEQ_BLK = 512


def _add_body(x_ref, p_ref, o_ref):
    o_ref[...] = x_ref[...] + p_ref[...]


def kernel(x, pos_table):
    B, L, D = x.shape
    S = _SEQ_BLK
    return pl.pallas_call(
        _add_body,
        grid=(L // S, B),
        in_specs=[
            pl.BlockSpec((1, S, D), lambda s, b: (b, s, 0)),
            pl.BlockSpec((S, D), lambda s, b: (s, 0)),
        ],
        out_specs=pl.BlockSpec((1, S, D), lambda s, b: (b, s, 0)),
        out_shape=jax.ShapeDtypeStruct(x.shape, x.dtype),
    )(x, pos_table)


# speedup vs baseline: 1.7186x; 1.1492x over previous
import jax
import jax.numpy as jnp
from jax.experimental import pallas as pl


_SEQ_BLK = 512


def _add_body(x_ref, p_ref, o_ref):
    o_ref[...] = x_ref[...] + p_ref[...]


def kernel(x, pos_table):
    B, L, D = x.shape
    S = _SEQ_BLK
    return pl.pallas_call(
        _add_body,
        grid=(L // S,),
        in_specs=[
            pl.BlockSpec((B, S, D), lambda s: (0, s, 0)),
            pl.BlockSpec((S, D), lambda s: (s, 0)),
        ],
        out_specs=pl.BlockSpec((B, S, D), lambda s: (0, s, 0)),
        out_shape=jax.ShapeDtypeStruct(x.shape, x.dtype),
    )(x, pos_table)
